# Initial kernel scaffold; baseline (speedup 1.0000x reference)
import jax
import jax.numpy as jnp
from jax.experimental import pallas as pl


def kernel(x, edge_index, W1, b1, W2, b2):
    raise NotImplementedError("write your pallas kernel here")


# speedup vs baseline: 2.3038x; 2.3038x over previous
import functools

import jax
import jax.numpy as jnp
from jax import lax
from jax.experimental import pallas as pl
from jax.experimental.pallas import tpu as pltpu
from jax.experimental.pallas import tpu_sc as plsc

NC = 2
NS = 16
LANES = 16
NW = NC * NS
CHUNK = 2000


@functools.lru_cache(maxsize=None)
def _make_segsum(n_feat, n_nodes, n_edges):
  rows_per_w = n_feat // NW
  mesh = plsc.VectorSubcoreMesh(core_axis_name="c", subcore_axis_name="s")

  @functools.partial(
      pl.kernel,
      mesh=mesh,
      compiler_params=pltpu.CompilerParams(
          use_tc_tiling_on_sc=False, needs_layout_passes=False),
      out_type=jax.ShapeDtypeStruct((n_feat * n_nodes,), jnp.float32),
      scratch_types=[
          pltpu.VMEM((rows_per_w * n_nodes,), jnp.float32),
          pltpu.VMEM((rows_per_w * n_nodes,), jnp.float32),
          pltpu.VMEM((CHUNK,), jnp.int32),
          pltpu.VMEM((CHUNK,), jnp.int32),
      ],
  )
  def segsum(xT_hbm, src_hbm, dst_hbm, out_hbm, xr, acc, sbuf, dbuf):
    wid = lax.axis_index("s") * NC + lax.axis_index("c")
    base = wid * rows_per_w * n_nodes
    pltpu.sync_copy(xT_hbm.at[pl.ds(base, rows_per_w * n_nodes)], xr)

    zeros16 = jnp.zeros((LANES,), jnp.float32)

    def zero_body(i, carry):
      acc[pl.ds(i * LANES, LANES)] = zeros16
      return carry

    lax.fori_loop(0, rows_per_w * n_nodes // LANES, zero_body, 0)

    def chunk_body(g, carry):
      pltpu.sync_copy(src_hbm.at[pl.ds(g * CHUNK, CHUNK)], sbuf)
      pltpu.sync_copy(dst_hbm.at[pl.ds(g * CHUNK, CHUNK)], dbuf)

      def grp(i, c2):
        s = sbuf[pl.ds(i * LANES, LANES)]
        d = dbuf[pl.ds(i * LANES, LANES)]
        for c in range(rows_per_w):
          off = jnp.full((LANES,), c * n_nodes, jnp.int32)
          v = plsc.load_gather(xr, [s + off])
          plsc.addupdate_scatter(acc, [d + off], v)
        return c2

      lax.fori_loop(0, CHUNK // LANES, grp, 0)
      return carry

    lax.fori_loop(0, n_edges // CHUNK, chunk_body, 0)
    pltpu.sync_copy(acc, out_hbm.at[pl.ds(base, rows_per_w * n_nodes)])

  return segsum


def _mm_relu(accT, W, b):
  f, n = accT.shape

  def body(a_ref, w_ref, b_ref, o_ref):
    o_ref[...] = jnp.maximum(
        jnp.dot(w_ref[...], a_ref[...], preferred_element_type=jnp.float32)
        + b_ref[...], 0.0)

  return pl.pallas_call(
      body,
      out_shape=jax.ShapeDtypeStruct((W.shape[0], n), jnp.float32),
  )(accT, W, b.reshape(-1, 1))


def _mm_out(accT, W, b):
  f, n = accT.shape
  c_out = W.shape[0]

  def body(a_ref, w_ref, b_ref, o_ref):
    o_ref[...] = lax.dot_general(
        a_ref[...], w_ref[...], (((0,), (1,)), ((), ())),
        preferred_element_type=jnp.float32) + b_ref[...]

  return pl.pallas_call(
      body,
      out_shape=jax.ShapeDtypeStruct((n, c_out), jnp.float32),
  )(accT, W, b.reshape(1, -1))


def kernel(x, edge_index, W1, b1, W2, b2):
  n_nodes, n_feat = x.shape
  n_edges = edge_index.shape[1]
  src = edge_index[0]
  dst = edge_index[1]
  xT = x.T

  segsum = _make_segsum(n_feat, n_nodes, n_edges)
  a1 = segsum(xT.reshape(-1), src, dst).reshape(n_feat, n_nodes)
  h1 = _mm_relu(a1, W1, b1)
  a2 = _make_segsum(h1.shape[0], n_nodes, n_edges)(
      h1.reshape(-1), src, dst).reshape(h1.shape[0], n_nodes)
  return _mm_out(a2, W2, b2)


# speedup vs baseline: 3.2376x; 1.4053x over previous
import functools

import jax
import jax.numpy as jnp
from jax import lax
from jax.experimental import pallas as pl
from jax.experimental.pallas import tpu as pltpu
from jax.experimental.pallas import tpu_sc as plsc

NC = 2
NS = 16
LANES = 16
NW = NC * NS
CHUNK = 2000


@functools.lru_cache(maxsize=None)
def _make_segsum(n_feat, n_nodes, n_edges):
  rows_per_w = n_feat // NW
  mesh = plsc.VectorSubcoreMesh(core_axis_name="c", subcore_axis_name="s")

  @functools.partial(
      pl.kernel,
      mesh=mesh,
      compiler_params=pltpu.CompilerParams(
          use_tc_tiling_on_sc=False, needs_layout_passes=False),
      out_type=jax.ShapeDtypeStruct((n_feat * n_nodes,), jnp.float32),
      scratch_types=[
          pltpu.VMEM((rows_per_w * n_nodes,), jnp.float32),
          pltpu.VMEM((rows_per_w * n_nodes,), jnp.float32),
          pltpu.VMEM((2, CHUNK), jnp.int32),
          pltpu.VMEM((2, CHUNK), jnp.int32),
          pltpu.SemaphoreType.DMA,
          pltpu.SemaphoreType.DMA,
          pltpu.SemaphoreType.DMA,
      ],
  )
  def segsum(xT_hbm, src_hbm, dst_hbm, out_hbm, xr, acc, sbuf, dbuf,
             sem0, sem1, xsem):
    wid = lax.axis_index("s") * NC + lax.axis_index("c")
    base = wid * rows_per_w * n_nodes
    nchunks = n_edges // CHUNK
    groups = CHUNK // LANES
    sems = (sem0, sem1)

    xcopy = pltpu.make_async_copy(
        xT_hbm.at[pl.ds(base, rows_per_w * n_nodes)], xr, xsem)
    xcopy.start()

    def start_chunk(b, g):
      pltpu.make_async_copy(
          src_hbm.at[pl.ds(g * CHUNK, CHUNK)], sbuf.at[b], sems[b]).start()
      pltpu.make_async_copy(
          dst_hbm.at[pl.ds(g * CHUNK, CHUNK)], dbuf.at[b], sems[b]).start()

    def wait_chunk(b):
      pltpu.make_async_copy(
          src_hbm.at[pl.ds(0, CHUNK)], sbuf.at[b], sems[b]).wait()
      pltpu.make_async_copy(
          dst_hbm.at[pl.ds(0, CHUNK)], dbuf.at[b], sems[b]).wait()

    start_chunk(0, 0)
    start_chunk(1, 1)

    zeros16 = jnp.zeros((LANES,), jnp.float32)

    def zero_body(i, carry):
      acc[pl.ds(i * LANES, LANES)] = zeros16
      return carry

    lax.fori_loop(0, rows_per_w * n_nodes // LANES, zero_body, 0,
                  unroll=8)
    xcopy.wait()

    def chunk_body(g, carry):
      for b in range(2):
        cidx = 2 * g + b
        wait_chunk(b)

        def grp(i, c2):
          s = sbuf[b, pl.ds(i * LANES, LANES)]
          d = dbuf[b, pl.ds(i * LANES, LANES)]
          for c in range(rows_per_w):
            off = jnp.full((LANES,), c * n_nodes, jnp.int32)
            v = plsc.load_gather(xr, [s + off])
            plsc.addupdate_scatter(acc, [d + off], v)
          return c2

        lax.fori_loop(0, groups, grp, 0, unroll=5)

        @pl.when(cidx + 2 < nchunks)
        def _():
          start_chunk(b, cidx + 2)
      return carry

    lax.fori_loop(0, nchunks // 2, chunk_body, 0)
    pltpu.sync_copy(acc, out_hbm.at[pl.ds(base, rows_per_w * n_nodes)])

  return segsum


def _mm_relu(accT, W, b):
  f, n = accT.shape

  def body(a_ref, w_ref, b_ref, o_ref):
    o_ref[...] = jnp.maximum(
        jnp.dot(w_ref[...], a_ref[...], preferred_element_type=jnp.float32)
        + b_ref[...], 0.0)

  return pl.pallas_call(
      body,
      out_shape=jax.ShapeDtypeStruct((W.shape[0], n), jnp.float32),
  )(accT, W, b.reshape(-1, 1))


def _mm_out(accT, W, b):
  f, n = accT.shape
  c_out = W.shape[0]

  def body(a_ref, w_ref, b_ref, o_ref):
    o_ref[...] = lax.dot_general(
        a_ref[...], w_ref[...], (((0,), (1,)), ((), ())),
        preferred_element_type=jnp.float32) + b_ref[...]

  return pl.pallas_call(
      body,
      out_shape=jax.ShapeDtypeStruct((n, c_out), jnp.float32),
  )(accT, W, b.reshape(1, -1))


def kernel(x, edge_index, W1, b1, W2, b2):
  n_nodes, n_feat = x.shape
  n_edges = edge_index.shape[1]
  src = edge_index[0]
  dst = edge_index[1]
  xT = x.T

  segsum = _make_segsum(n_feat, n_nodes, n_edges)
  a1 = segsum(xT.reshape(-1), src, dst).reshape(n_feat, n_nodes)
  h1 = _mm_relu(a1, W1, b1)
  a2 = _make_segsum(h1.shape[0], n_nodes, n_edges)(
      h1.reshape(-1), src, dst).reshape(h1.shape[0], n_nodes)
  return _mm_out(a2, W2, b2)


# speedup vs baseline: 7.3083x; 2.2573x over previous
import functools

import jax
import jax.numpy as jnp
from jax import lax
from jax.experimental import pallas as pl
from jax.experimental.pallas import tpu as pltpu
from jax.experimental.pallas import tpu_sc as plsc

NC = 2
NS = 16
LANES = 16
NW = NC * NS
CHUNK = 2000
UNROLL = 5


@functools.lru_cache(maxsize=None)
def _make_segsum(n_feat, n_nodes, n_edges):
  rows_per_w = n_feat // NW
  mesh = plsc.VectorSubcoreMesh(core_axis_name="c", subcore_axis_name="s")

  @functools.partial(
      pl.kernel,
      mesh=mesh,
      compiler_params=pltpu.CompilerParams(
          use_tc_tiling_on_sc=False, needs_layout_passes=False),
      out_type=jax.ShapeDtypeStruct((n_feat * n_nodes,), jnp.float32),
      scratch_types=[
          pltpu.VMEM((rows_per_w * n_nodes,), jnp.float32),
          pltpu.VMEM((rows_per_w * n_nodes,), jnp.float32),
          pltpu.VMEM((2, CHUNK), jnp.int32),
          pltpu.VMEM((2, CHUNK), jnp.int32),
          pltpu.SemaphoreType.DMA,
          pltpu.SemaphoreType.DMA,
          pltpu.SemaphoreType.DMA,
      ],
  )
  def segsum(xT_hbm, src_hbm, dst_hbm, out_hbm, xr, acc, sbuf, dbuf,
             sem0, sem1, xsem):
    wid = lax.axis_index("s") * NC + lax.axis_index("c")
    base = wid * rows_per_w * n_nodes
    nchunks = n_edges // CHUNK
    groups = CHUNK // LANES
    sems = (sem0, sem1)

    xcopy = pltpu.make_async_copy(
        xT_hbm.at[pl.ds(base, rows_per_w * n_nodes)], xr, xsem)
    xcopy.start()

    def start_chunk(b, g):
      pltpu.make_async_copy(
          src_hbm.at[pl.ds(g * CHUNK, CHUNK)], sbuf.at[b], sems[b]).start()
      pltpu.make_async_copy(
          dst_hbm.at[pl.ds(g * CHUNK, CHUNK)], dbuf.at[b], sems[b]).start()

    def wait_chunk(b):
      pltpu.make_async_copy(
          src_hbm.at[pl.ds(0, CHUNK)], sbuf.at[b], sems[b]).wait()
      pltpu.make_async_copy(
          dst_hbm.at[pl.ds(0, CHUNK)], dbuf.at[b], sems[b]).wait()

    start_chunk(0, 0)
    start_chunk(1, 1)

    zeros16 = jnp.zeros((LANES,), jnp.float32)

    def zero_body(i, carry):
      acc[pl.ds(i * LANES, LANES)] = zeros16
      return carry

    lax.fori_loop(0, rows_per_w * n_nodes // LANES, zero_body, 0,
                  unroll=8)
    xcopy.wait()

    def chunk_body(g, carry):
      for b in range(2):
        cidx = 2 * g + b
        wait_chunk(b)

        def grp(i, c2):
          pending = []
          for u in range(UNROLL):
            j = i * UNROLL + u
            s = sbuf[b, pl.ds(j * LANES, LANES)]
            d = dbuf[b, pl.ds(j * LANES, LANES)]
            for c in range(rows_per_w):
              off = jnp.full((LANES,), c * n_nodes, jnp.int32)
              pending.append((d + off, plsc.load_gather(xr, [s + off])))
          for dd, v in pending:
            plsc.addupdate_scatter(acc, [dd], v)
          return c2

        lax.fori_loop(0, groups // UNROLL, grp, 0)

        @pl.when(cidx + 2 < nchunks)
        def _():
          start_chunk(b, cidx + 2)
      return carry

    lax.fori_loop(0, nchunks // 2, chunk_body, 0)
    pltpu.sync_copy(acc, out_hbm.at[pl.ds(base, rows_per_w * n_nodes)])

  return segsum


def _mm_relu(accT, W, b):
  f, n = accT.shape

  def body(a_ref, w_ref, b_ref, o_ref):
    o_ref[...] = jnp.maximum(
        jnp.dot(w_ref[...], a_ref[...], preferred_element_type=jnp.float32)
        + b_ref[...], 0.0)

  return pl.pallas_call(
      body,
      out_shape=jax.ShapeDtypeStruct((W.shape[0], n), jnp.float32),
  )(accT, W, b.reshape(-1, 1))


def _mm_out(accT, W, b):
  f, n = accT.shape
  c_out = W.shape[0]

  def body(a_ref, w_ref, b_ref, o_ref):
    o_ref[...] = lax.dot_general(
        a_ref[...], w_ref[...], (((0,), (1,)), ((), ())),
        preferred_element_type=jnp.float32) + b_ref[...]

  return pl.pallas_call(
      body,
      out_shape=jax.ShapeDtypeStruct((n, c_out), jnp.float32),
  )(accT, W, b.reshape(1, -1))


def kernel(x, edge_index, W1, b1, W2, b2):
  n_nodes, n_feat = x.shape
  n_edges = edge_index.shape[1]
  src = edge_index[0]
  dst = edge_index[1]
  xT = x.T

  segsum = _make_segsum(n_feat, n_nodes, n_edges)
  a1 = segsum(xT.reshape(-1), src, dst).reshape(n_feat, n_nodes)
  h1 = _mm_relu(a1, W1, b1)
  a2 = _make_segsum(h1.shape[0], n_nodes, n_edges)(
      h1.reshape(-1), src, dst).reshape(h1.shape[0], n_nodes)
  return _mm_out(a2, W2, b2)


# speedup vs baseline: 8.7245x; 1.1938x over previous
import functools

import jax
import jax.numpy as jnp
from jax import lax
from jax.experimental import pallas as pl
from jax.experimental.pallas import tpu as pltpu
from jax.experimental.pallas import tpu_sc as plsc

NC = 2
NS = 16
LANES = 16
NW = NC * NS
CHUNK = 2000
UNROLL = 3


@functools.lru_cache(maxsize=None)
def _make_segsum(n_feat, n_nodes, n_edges):
  rows_per_w = n_feat // NW
  mesh = plsc.VectorSubcoreMesh(core_axis_name="c", subcore_axis_name="s")

  @functools.partial(
      pl.kernel,
      mesh=mesh,
      compiler_params=pltpu.CompilerParams(
          use_tc_tiling_on_sc=False, needs_layout_passes=False),
      out_type=jax.ShapeDtypeStruct((n_feat * n_nodes,), jnp.float32),
      scratch_types=[
          pltpu.VMEM((rows_per_w // 2 * n_nodes,), jnp.int32),
          pltpu.VMEM((rows_per_w * n_nodes,), jnp.float32),
          pltpu.VMEM((2, CHUNK), jnp.int32),
          pltpu.SemaphoreType.DMA,
          pltpu.SemaphoreType.DMA,
          pltpu.SemaphoreType.DMA,
      ],
  )
  def segsum(xpk_hbm, edges_hbm, out_hbm, xr, acc, ebuf, sem0, sem1, xsem):
    wid = lax.axis_index("s") * NC + lax.axis_index("c")
    base = wid * rows_per_w * n_nodes
    pairs = rows_per_w // 2
    nchunks = n_edges // CHUNK
    groups = CHUNK // LANES
    sems = (sem0, sem1)

    xcopy = pltpu.make_async_copy(
        xpk_hbm.at[pl.ds(wid * pairs * n_nodes, pairs * n_nodes)], xr, xsem)
    xcopy.start()

    def start_chunk(b, g):
      pltpu.make_async_copy(
          edges_hbm.at[pl.ds(g * CHUNK, CHUNK)], ebuf.at[b], sems[b]).start()

    def wait_chunk(b):
      pltpu.make_async_copy(
          edges_hbm.at[pl.ds(0, CHUNK)], ebuf.at[b], sems[b]).wait()

    start_chunk(0, 0)
    start_chunk(1, 1)

    zeros16 = jnp.zeros((LANES,), jnp.float32)

    def zero_body(i, carry):
      acc[pl.ds(i * LANES, LANES)] = zeros16
      return carry

    lax.fori_loop(0, rows_per_w * n_nodes // LANES, zero_body, 0,
                  unroll=8)
    xcopy.wait()

    def chunk_body(g, carry):
      for b in range(2):
        cidx = 2 * g + b
        wait_chunk(b)

        @plsc.parallel_loop(0, groups, unroll=UNROLL)
        def _(j):
          e = ebuf[b, pl.ds(j * LANES, LANES)]
          s = e & jnp.int32(0xFFFF)
          d = lax.shift_right_logical(e, jnp.int32(16))
          for p in range(pairs):
            poff = jnp.full((LANES,), p * n_nodes, jnp.int32)
            w = plsc.load_gather(xr, [s + poff])
            vlo = plsc.bitcast(w << jnp.int32(16), jnp.float32)
            vhi = plsc.bitcast(w & jnp.int32(-65536), jnp.float32)
            olo = jnp.full((LANES,), 2 * p * n_nodes, jnp.int32)
            ohi = jnp.full((LANES,), (2 * p + 1) * n_nodes, jnp.int32)
            plsc.addupdate_scatter(acc, [d + olo], vlo)
            plsc.addupdate_scatter(acc, [d + ohi], vhi)

        @pl.when(cidx + 2 < nchunks)
        def _():
          start_chunk(b, cidx + 2)
      return carry

    lax.fori_loop(0, nchunks // 2, chunk_body, 0)
    pltpu.sync_copy(acc, out_hbm.at[pl.ds(base, rows_per_w * n_nodes)])

  return segsum


def _bf16_pack(lo_f32, hi_f32):
  lo = lax.bitcast_convert_type(
      lo_f32.astype(jnp.bfloat16).astype(jnp.float32), jnp.int32)
  hi = lax.bitcast_convert_type(
      hi_f32.astype(jnp.bfloat16).astype(jnp.float32), jnp.int32)
  return lax.shift_right_logical(lo, 16) | (hi & jnp.int32(-65536))


def _prep(edge_index, x_even, x_odd):
  n_edges = edge_index.shape[1]
  pr, n = x_even.shape

  def body(e_ref, xe_ref, xo_ref, eo_ref, xpk_ref):
    eo_ref[...] = e_ref[0, :] | (e_ref[1, :] << 16)
    xpk_ref[...] = _bf16_pack(xe_ref[...], xo_ref[...])

  return pl.pallas_call(
      body,
      out_shape=(
          jax.ShapeDtypeStruct((n_edges,), jnp.int32),
          jax.ShapeDtypeStruct((pr, n), jnp.int32),
      ),
  )(edge_index, x_even, x_odd)


def _mm_relu_pack(accT, We, Wo, be, bo):
  f, n = accT.shape

  def body(a_ref, we_ref, wo_ref, be_ref, bo_ref, o_ref):
    a = a_ref[...]
    he = jnp.maximum(
        jnp.dot(we_ref[...], a, preferred_element_type=jnp.float32)
        + be_ref[...], 0.0)
    ho = jnp.maximum(
        jnp.dot(wo_ref[...], a, preferred_element_type=jnp.float32)
        + bo_ref[...], 0.0)
    o_ref[...] = _bf16_pack(he, ho)

  return pl.pallas_call(
      body,
      out_shape=jax.ShapeDtypeStruct((We.shape[0], n), jnp.int32),
  )(accT, We, Wo, be.reshape(-1, 1), bo.reshape(-1, 1))


def _mm_out(accT, W, b):
  f, n = accT.shape
  c_out = W.shape[0]

  def body(a_ref, w_ref, b_ref, o_ref):
    o_ref[...] = lax.dot_general(
        a_ref[...], w_ref[...], (((0,), (1,)), ((), ())),
        preferred_element_type=jnp.float32) + b_ref[...]

  return pl.pallas_call(
      body,
      out_shape=jax.ShapeDtypeStruct((n, c_out), jnp.float32),
  )(accT, W, b.reshape(1, -1))


def kernel(x, edge_index, W1, b1, W2, b2):
  n_nodes, n_feat = x.shape
  n_edges = edge_index.shape[1]
  xT = x.T
  edges, xpk = _prep(edge_index, xT[0::2], xT[1::2])

  segsum = _make_segsum(n_feat, n_nodes, n_edges)
  a1 = segsum(xpk.reshape(-1), edges).reshape(n_feat, n_nodes)
  h1pk = _mm_relu_pack(a1, W1[0::2], W1[1::2], b1[0::2], b1[1::2])
  a2 = segsum(h1pk.reshape(-1), edges).reshape(n_feat, n_nodes)
  return _mm_out(a2, W2, b2)
